# Initial kernel scaffold; baseline (speedup 1.0000x reference)
import jax
import jax.numpy as jnp
from jax.experimental import pallas as pl


def kernel(x, edge_index, batch, W1, b1, W2, b2, W3, b3, Wl, bl):
    raise NotImplementedError("write your pallas kernel here")


# speedup vs baseline: 11.0096x; 11.0096x over previous
import functools

import jax
import jax.numpy as jnp
from jax import lax
from jax.experimental import pallas as pl
from jax.experimental.pallas import tpu as pltpu
from jax.experimental.pallas import tpu_sc as plsc

N = 10000
E = 320000
F_IN = 128
H = 64
C = 10
G = 64

NP = 10240
ROWS_PER_TILE = NP // 16
EP = 327680
EDGE_ROWS = EP // 128
TILE_EDGE_ROWS = EDGE_ROWS // 32
CHUNK_ROWS = 8
NCHUNK = TILE_EDGE_ROWS // CHUNK_ROWS

_mesh = functools.partial(
    plsc.VectorSubcoreMesh, core_axis_name="c", subcore_axis_name="s")
_sc_params = pltpu.CompilerParams(use_tc_tiling_on_sc=False)


@functools.partial(
    pl.kernel,
    out_type=jax.ShapeDtypeStruct((2 * NP,), jnp.float32),
    mesh=_mesh(),
    compiler_params=_sc_params,
    scratch_types=[
        pltpu.VMEM((CHUNK_ROWS, 128), jnp.int32),
        pltpu.VMEM((128,), jnp.float32),
        pltpu.VMEM_SHARED((NP,), jnp.float32),
    ],
)
def _sc_degree(dst_hbm, ones_hbm, zeros_hbm, out_hbm, didx, ones_v, sh_deg):
  cid = lax.axis_index("c")
  sid = lax.axis_index("s")
  tid = cid * 16 + sid
  roff = sid * ROWS_PER_TILE
  pltpu.sync_copy(ones_hbm, ones_v)
  pltpu.sync_copy(zeros_hbm.at[pl.ds(roff, ROWS_PER_TILE)],
                  sh_deg.at[pl.ds(roff, ROWS_PER_TILE)])
  plsc.subcore_barrier()

  base = tid * TILE_EDGE_ROWS

  def chunk(c, carry):
    so = base + c * CHUNK_ROWS
    pltpu.sync_copy(dst_hbm.at[pl.ds(so, CHUNK_ROWS)], didx)
    for j in range(CHUNK_ROWS):
      pltpu.sync_copy(ones_v, sh_deg.at[didx.at[j]], add=True)
    return carry

  lax.fori_loop(0, NCHUNK, chunk, 0)
  plsc.subcore_barrier()
  pltpu.sync_copy(sh_deg.at[pl.ds(roff, ROWS_PER_TILE)],
                  out_hbm.at[pl.ds(cid * NP + roff, ROWS_PER_TILE)])


@functools.partial(
    pl.kernel,
    out_type=jax.ShapeDtypeStruct((2, NP, H), jnp.float32),
    mesh=_mesh(),
    compiler_params=_sc_params,
    scratch_types=[
        pltpu.VMEM((CHUNK_ROWS, 128), jnp.int32),
        pltpu.VMEM((CHUNK_ROWS, 128), jnp.int32),
        pltpu.VMEM((CHUNK_ROWS * 128, H), jnp.float32),
        pltpu.VMEM_SHARED((NP, H), jnp.float32),
    ],
)
def _sc_layer(y_hbm, src_hbm, dst_hbm, zeros_hbm, out_hbm,
              sidx, didx, rows, sh_z):
  cid = lax.axis_index("c")
  sid = lax.axis_index("s")
  tid = cid * 16 + sid
  roff = sid * ROWS_PER_TILE
  pltpu.sync_copy(zeros_hbm.at[pl.ds(roff, ROWS_PER_TILE)],
                  sh_z.at[pl.ds(roff, ROWS_PER_TILE)])
  plsc.subcore_barrier()

  base = tid * TILE_EDGE_ROWS

  def chunk(c, carry):
    so = base + c * CHUNK_ROWS
    pltpu.sync_copy(src_hbm.at[pl.ds(so, CHUNK_ROWS)], sidx)
    pltpu.sync_copy(dst_hbm.at[pl.ds(so, CHUNK_ROWS)], didx)
    for j in range(CHUNK_ROWS):
      pltpu.sync_copy(y_hbm.at[sidx.at[j]], rows.at[pl.ds(j * 128, 128)])
    for j in range(CHUNK_ROWS):
      pltpu.sync_copy(rows.at[pl.ds(j * 128, 128)], sh_z.at[didx.at[j]],
                      add=True)
    return carry

  lax.fori_loop(0, NCHUNK, chunk, 0)
  plsc.subcore_barrier()
  pltpu.sync_copy(sh_z.at[pl.ds(roff, ROWS_PER_TILE)],
                  out_hbm.at[cid, pl.ds(roff, ROWS_PER_TILE)])


def _tc_pre_body(deg_ref, x_ref, w1_ref, dinv_ref, y1_ref):
  deg = deg_ref[:, 0:1] + deg_ref[:, 1:2] + 1.0
  dinv = lax.rsqrt(deg)
  dinv_ref[...] = dinv
  xw = jnp.dot(x_ref[...], w1_ref[...], preferred_element_type=jnp.float32)
  y1_ref[...] = xw * dinv


def _tc_pre(deg_t, x_p, w1):
  return pl.pallas_call(
      _tc_pre_body,
      out_shape=(jax.ShapeDtypeStruct((NP, 1), jnp.float32),
                 jax.ShapeDtypeStruct((NP, H), jnp.float32)),
  )(deg_t, x_p, w1)


def _tc_mid_body(z_ref, y_ref, dinv_ref, b_ref, w_ref, ynext_ref):
  z = z_ref[0] + z_ref[1] + y_ref[...]
  dinv = dinv_ref[...]
  h = jnp.maximum(z * dinv + b_ref[...], 0.0)
  ynext_ref[...] = jnp.dot(
      h, w_ref[...], preferred_element_type=jnp.float32) * dinv


def _tc_mid(z_parts, y_prev, dinv, b, w_next):
  return pl.pallas_call(
      _tc_mid_body,
      out_shape=jax.ShapeDtypeStruct((NP, H), jnp.float32),
  )(z_parts, y_prev, dinv, b, w_next)


def _tc_final_body(z_ref, y_ref, dinv_ref, b_ref, batch_ref, wl_ref, bl_ref,
                   out_ref, hpool_ref):
  z = z_ref[0] + z_ref[1] + y_ref[...]
  h = z * dinv_ref[...] + b_ref[...]
  h_n = h[:N, :]
  gids = lax.broadcasted_iota(jnp.int32, (G, N), 0)
  seg = (batch_ref[...] == gids).astype(jnp.float32)
  sums = jnp.dot(seg, h_n, preferred_element_type=jnp.float32)
  cnt = jnp.sum(seg, axis=1, keepdims=True)
  hpool = sums / jnp.maximum(cnt, 1.0)
  hpool_ref[...] = hpool
  out_ref[...] = jnp.dot(
      hpool, wl_ref[...], preferred_element_type=jnp.float32) + bl_ref[...]


def _tc_final(z_parts, y3, dinv, b3, batch2, wl, bl):
  return pl.pallas_call(
      _tc_final_body,
      out_shape=(jax.ShapeDtypeStruct((G, C), jnp.float32),
                 jax.ShapeDtypeStruct((G, H), jnp.float32)),
  )(z_parts, y3, dinv, b3, batch2, wl, bl)


def kernel(x, edge_index, batch, W1, b1, W2, b2, W3, b3, Wl, bl):
  src = edge_index[0]
  dst = edge_index[1]
  pad = jnp.full((EP - E,), N, dtype=jnp.int32)
  src2 = jnp.concatenate([src, pad]).reshape(EDGE_ROWS, 128)
  dst2 = jnp.concatenate([dst, pad]).reshape(EDGE_ROWS, 128)
  x_p = jnp.pad(x, ((0, NP - N), (0, 0)))
  batch2 = batch.reshape(1, N)

  zeros_n = jnp.zeros((NP,), jnp.float32)
  zeros_nh = jnp.zeros((NP, H), jnp.float32)
  ones128 = jnp.ones((128,), jnp.float32)

  deg_parts = _sc_degree(dst2, ones128, zeros_n)
  deg_t = jnp.transpose(deg_parts.reshape(2, NP))
  dinv, y1 = _tc_pre(deg_t, x_p, W1)

  z1 = _sc_layer(y1, src2, dst2, zeros_nh)
  y2 = _tc_mid(z1, y1, dinv, b1.reshape(1, H), W2)

  z2 = _sc_layer(y2, src2, dst2, zeros_nh)
  y3 = _tc_mid(z2, y2, dinv, b2.reshape(1, H), W3)

  z3 = _sc_layer(y3, src2, dst2, zeros_nh)
  out, hpool = _tc_final(z3, y3, dinv, b3.reshape(1, H), batch2,
                         Wl, bl.reshape(1, C))
  return (out, hpool)


# speedup vs baseline: 12.4479x; 1.1306x over previous
import functools

import jax
import jax.numpy as jnp
from jax import lax
from jax.experimental import pallas as pl
from jax.experimental.pallas import tpu as pltpu
from jax.experimental.pallas import tpu_sc as plsc

N = 10000
E = 320000
F_IN = 128
H = 64
C = 10
G = 64

NP = 10240
ROWS_PER_TILE = NP // 16
EP = 327680
EDGE_ROWS = EP // 128
TILE_EDGE_ROWS = EDGE_ROWS // 32
CHUNK_ROWS = 8
NCHUNK = TILE_EDGE_ROWS // CHUNK_ROWS

_mesh = functools.partial(
    plsc.VectorSubcoreMesh, core_axis_name="c", subcore_axis_name="s")
_sc_params = pltpu.CompilerParams(use_tc_tiling_on_sc=False)


@functools.partial(
    pl.kernel,
    out_type=jax.ShapeDtypeStruct((2 * NP,), jnp.float32),
    mesh=_mesh(),
    compiler_params=_sc_params,
    scratch_types=[
        pltpu.VMEM((CHUNK_ROWS, 128), jnp.int32),
        pltpu.VMEM((128,), jnp.float32),
        pltpu.VMEM_SHARED((NP,), jnp.float32),
    ],
)
def _sc_degree(dst_hbm, ones_hbm, zeros_hbm, out_hbm, didx, ones_v, sh_deg):
  cid = lax.axis_index("c")
  sid = lax.axis_index("s")
  tid = cid * 16 + sid
  roff = sid * ROWS_PER_TILE
  pltpu.sync_copy(ones_hbm, ones_v)
  pltpu.sync_copy(zeros_hbm.at[pl.ds(roff, ROWS_PER_TILE)],
                  sh_deg.at[pl.ds(roff, ROWS_PER_TILE)])
  plsc.subcore_barrier()

  base = tid * TILE_EDGE_ROWS

  def chunk(c, carry):
    so = base + c * CHUNK_ROWS
    pltpu.sync_copy(dst_hbm.at[pl.ds(so, CHUNK_ROWS)], didx)
    for j in range(CHUNK_ROWS):
      pltpu.sync_copy(ones_v, sh_deg.at[didx.at[j]], add=True)
    return carry

  lax.fori_loop(0, NCHUNK, chunk, 0)
  plsc.subcore_barrier()
  pltpu.sync_copy(sh_deg.at[pl.ds(roff, ROWS_PER_TILE)],
                  out_hbm.at[pl.ds(cid * NP + roff, ROWS_PER_TILE)])


@functools.partial(
    pl.kernel,
    out_type=jax.ShapeDtypeStruct((2, NP, H), jnp.float32),
    mesh=_mesh(),
    compiler_params=_sc_params,
    scratch_types=[
        pltpu.VMEM((CHUNK_ROWS, 128), jnp.int32),
        pltpu.VMEM((CHUNK_ROWS, 128), jnp.int32),
        pltpu.VMEM((CHUNK_ROWS * 128, H), jnp.float32),
        pltpu.VMEM_SHARED((NP, H), jnp.float32),
        pltpu.SemaphoreType.DMA,
        pltpu.SemaphoreType.DMA,
    ],
)
def _sc_layer(y_hbm, src_hbm, dst_hbm, zeros_hbm, out_hbm,
              sidx, didx, rows, sh_z, gsem, ssem):
  cid = lax.axis_index("c")
  sid = lax.axis_index("s")
  tid = cid * 16 + sid
  roff = sid * ROWS_PER_TILE
  pltpu.sync_copy(zeros_hbm.at[pl.ds(roff, ROWS_PER_TILE)],
                  sh_z.at[pl.ds(roff, ROWS_PER_TILE)])
  plsc.subcore_barrier()

  base = tid * TILE_EDGE_ROWS

  def chunk(c, carry):
    @pl.when(c > 0)
    def _():
      for j in range(CHUNK_ROWS):
        pltpu.make_async_copy(rows.at[pl.ds(j * 128, 128)],
                              sh_z.at[didx.at[j]], ssem).wait()
    so = base + c * CHUNK_ROWS
    pltpu.sync_copy(src_hbm.at[pl.ds(so, CHUNK_ROWS)], sidx)
    pltpu.sync_copy(dst_hbm.at[pl.ds(so, CHUNK_ROWS)], didx)
    gathers = [
        pltpu.async_copy(y_hbm.at[sidx.at[j]], rows.at[pl.ds(j * 128, 128)],
                         gsem)
        for j in range(CHUNK_ROWS)
    ]
    for j in range(CHUNK_ROWS):
      gathers[j].wait()
      pltpu.async_copy(rows.at[pl.ds(j * 128, 128)], sh_z.at[didx.at[j]],
                       ssem, add=True)
    return carry

  lax.fori_loop(0, NCHUNK, chunk, 0)
  for j in range(CHUNK_ROWS):
    pltpu.make_async_copy(rows.at[pl.ds(j * 128, 128)],
                          sh_z.at[didx.at[j]], ssem).wait()
  plsc.subcore_barrier()
  pltpu.sync_copy(sh_z.at[pl.ds(roff, ROWS_PER_TILE)],
                  out_hbm.at[cid, pl.ds(roff, ROWS_PER_TILE)])


def _tc_pre_body(deg_ref, x_ref, w1_ref, dinv_ref, y1_ref):
  deg = deg_ref[:, 0:1] + deg_ref[:, 1:2] + 1.0
  dinv = lax.rsqrt(deg)
  dinv_ref[...] = dinv
  xw = jnp.dot(x_ref[...], w1_ref[...], preferred_element_type=jnp.float32)
  y1_ref[...] = xw * dinv


def _tc_pre(deg_t, x_p, w1):
  return pl.pallas_call(
      _tc_pre_body,
      out_shape=(jax.ShapeDtypeStruct((NP, 1), jnp.float32),
                 jax.ShapeDtypeStruct((NP, H), jnp.float32)),
  )(deg_t, x_p, w1)


def _tc_mid_body(z_ref, y_ref, dinv_ref, b_ref, w_ref, ynext_ref):
  z = z_ref[0] + z_ref[1] + y_ref[...]
  dinv = dinv_ref[...]
  h = jnp.maximum(z * dinv + b_ref[...], 0.0)
  ynext_ref[...] = jnp.dot(
      h, w_ref[...], preferred_element_type=jnp.float32) * dinv


def _tc_mid(z_parts, y_prev, dinv, b, w_next):
  return pl.pallas_call(
      _tc_mid_body,
      out_shape=jax.ShapeDtypeStruct((NP, H), jnp.float32),
  )(z_parts, y_prev, dinv, b, w_next)


def _tc_final_body(z_ref, y_ref, dinv_ref, b_ref, batch_ref, wl_ref, bl_ref,
                   out_ref, hpool_ref):
  z = z_ref[0] + z_ref[1] + y_ref[...]
  h = z * dinv_ref[...] + b_ref[...]
  h_n = h[:N, :]
  gids = lax.broadcasted_iota(jnp.int32, (G, N), 0)
  seg = (batch_ref[...] == gids).astype(jnp.float32)
  sums = jnp.dot(seg, h_n, preferred_element_type=jnp.float32)
  cnt = jnp.sum(seg, axis=1, keepdims=True)
  hpool = sums / jnp.maximum(cnt, 1.0)
  hpool_ref[...] = hpool
  out_ref[...] = jnp.dot(
      hpool, wl_ref[...], preferred_element_type=jnp.float32) + bl_ref[...]


def _tc_final(z_parts, y3, dinv, b3, batch2, wl, bl):
  return pl.pallas_call(
      _tc_final_body,
      out_shape=(jax.ShapeDtypeStruct((G, C), jnp.float32),
                 jax.ShapeDtypeStruct((G, H), jnp.float32)),
  )(z_parts, y3, dinv, b3, batch2, wl, bl)


def kernel(x, edge_index, batch, W1, b1, W2, b2, W3, b3, Wl, bl):
  src = edge_index[0]
  dst = edge_index[1]
  pad = jnp.full((EP - E,), N, dtype=jnp.int32)
  src2 = jnp.concatenate([src, pad]).reshape(EDGE_ROWS, 128)
  dst2 = jnp.concatenate([dst, pad]).reshape(EDGE_ROWS, 128)
  x_p = jnp.pad(x, ((0, NP - N), (0, 0)))
  batch2 = batch.reshape(1, N)

  zeros_n = jnp.zeros((NP,), jnp.float32)
  zeros_nh = jnp.zeros((NP, H), jnp.float32)
  ones128 = jnp.ones((128,), jnp.float32)

  deg_parts = _sc_degree(dst2, ones128, zeros_n)
  deg_t = jnp.transpose(deg_parts.reshape(2, NP))
  dinv, y1 = _tc_pre(deg_t, x_p, W1)

  z1 = _sc_layer(y1, src2, dst2, zeros_nh)
  y2 = _tc_mid(z1, y1, dinv, b1.reshape(1, H), W2)

  z2 = _sc_layer(y2, src2, dst2, zeros_nh)
  y3 = _tc_mid(z2, y2, dinv, b2.reshape(1, H), W3)

  z3 = _sc_layer(y3, src2, dst2, zeros_nh)
  out, hpool = _tc_final(z3, y3, dinv, b3.reshape(1, H), batch2,
                         Wl, bl.reshape(1, C))
  return (out, hpool)
